# baseline (speedup 1.0000x reference)
import jax
import jax.numpy as jnp
from jax import lax
from jax.experimental import pallas as pl
from jax.experimental.pallas import tpu as pltpu
from jax.experimental.pallas import tpu_sc as plsc

_NLO = 1.0000500033334732e-04
_NHI = 9.210340371976184
_LN2 = 0.6931471805599453

_LOGC = (
    -0.006151470821350813, 0.08406148105859756, -0.5094411969184875,
    1.8016612529754639, -4.118063449859619, 6.3783440589904785,
    -6.894534587860107, 5.635407447814941, -2.371283531188965,
)

_PIX = 512 * 512
_SC_NB = 4
_SC_NC, _SC_NS = 2, 16
_SC_NW = _SC_NC * _SC_NS
_SC_CH = 4096


def _tc_body(x_ref, t_ref, roi_ref, out_ref, acc_ref):
    i = pl.program_id(0)

    @pl.when(i == 0)
    def _init():
        acc_ref[0] = 0.0
        acc_ref[1] = 0.0

    x = x_ref[...]
    x0, x1, x2, x3 = x[:, 0], x[:, 1], x[:, 2], x[:, 3]
    s = (jnp.exp(x0) + jnp.exp(x1)) + (jnp.exp(x2) + jnp.exp(x3))

    t = t_ref[...]
    xt = jnp.where(t == 0, x0, jnp.where(t == 1, x1, jnp.where(t == 2, x2, x3)))
    nll = jnp.clip(jnp.log(s) - xt, _NLO, _NHI)

    live = roi_ref[...] != 0
    acc_ref[0] += jnp.sum(jnp.where(live, nll, 0.0))
    acc_ref[1] += jnp.sum(jnp.where(live, 1.0, 0.0))

    @pl.when(i == pl.num_programs(0) - 1)
    def _fin():
        out_ref[0, 0] = acc_ref[0]
        out_ref[0, 1] = acc_ref[1]


def _tc_partial(input, target, ROI, nb_skip):
    B, C, H, W = input.shape
    NB = 2
    grid = ((B - nb_skip) // NB,)
    off = nb_skip // NB
    out = pl.pallas_call(
        _tc_body,
        grid=grid,
        in_specs=[
            pl.BlockSpec((NB, C, H, W), lambda i: (i + off, 0, 0, 0)),
            pl.BlockSpec((NB, H, W), lambda i: (i + off, 0, 0)),
            pl.BlockSpec((NB, H, W), lambda i: (i + off, 0, 0)),
        ],
        out_specs=pl.BlockSpec((1, 2), lambda i: (0, 0), memory_space=pltpu.SMEM),
        out_shape=jax.ShapeDtypeStruct((1, 2), jnp.float32),
        scratch_shapes=[pltpu.SMEM((2,), jnp.float32)],
    )(input, target, ROI)
    return out[0, 0], out[0, 1]


def _sc_body(x_hbm, t_hbm, r_hbm, num_out, den_out,
             x0b, x1b, x2b, x3b, tb, rb, nacc, dacc):
    wid = lax.axis_index("s") * _SC_NC + lax.axis_index("c")
    pix_per_w = _SC_NB * _PIX // _SC_NW
    n_chunks = pix_per_w // _SC_CH

    def step(i, acc):
        an, ad = acc
        sl = pl.ds(i * 16, 16)
        x0 = x0b[sl]
        x1 = x1b[sl]
        x2 = x2b[sl]
        x3 = x3b[sl]
        s = (jnp.exp(x0) + jnp.exp(x1)) + (jnp.exp(x2) + jnp.exp(x3))
        bits = lax.bitcast_convert_type(s, jnp.int32)
        e = jnp.right_shift(bits, 23) - 127
        mb = jnp.bitwise_or(jnp.bitwise_and(bits, 0x007FFFFF), 0x3F800000)
        m = lax.bitcast_convert_type(mb, jnp.float32)
        p = jnp.full((16,), _LOGC[0], jnp.float32)
        for c in _LOGC[1:]:
            p = p * m + c
        logs = e.astype(jnp.float32) * _LN2 + p
        t = tb[sl]
        xt = jnp.where(t == 0, x0, jnp.where(t == 1, x1, jnp.where(t == 2, x2, x3)))
        nll = jnp.clip(logs - xt, _NLO, _NHI)
        live = rb[sl] != 0
        an = an + jnp.where(live, nll, 0.0)
        ad = ad + jnp.where(live, 1.0, 0.0)
        return (an, ad)

    acc = (jnp.zeros((16,), jnp.float32), jnp.zeros((16,), jnp.float32))
    for j in range(n_chunks):
        gp = wid * pix_per_w + j * _SC_CH
        b = gp // _PIX
        poff = gp - b * _PIX
        xbase = b * (4 * _PIX) + poff
        pltpu.sync_copy(x_hbm.at[pl.ds(xbase, _SC_CH)], x0b)
        pltpu.sync_copy(x_hbm.at[pl.ds(xbase + _PIX, _SC_CH)], x1b)
        pltpu.sync_copy(x_hbm.at[pl.ds(xbase + 2 * _PIX, _SC_CH)], x2b)
        pltpu.sync_copy(x_hbm.at[pl.ds(xbase + 3 * _PIX, _SC_CH)], x3b)
        pltpu.sync_copy(t_hbm.at[pl.ds(gp, _SC_CH)], tb)
        pltpu.sync_copy(r_hbm.at[pl.ds(gp, _SC_CH)], rb)
        acc = lax.fori_loop(0, _SC_CH // 16, step, acc)

    nacc[...] = acc[0]
    dacc[...] = acc[1]
    pltpu.sync_copy(nacc, num_out.at[wid])
    pltpu.sync_copy(dacc, den_out.at[wid])


def _sc_partial(x_flat, t_flat, r_flat):
    f = pl.kernel(
        _sc_body,
        out_type=[
            jax.ShapeDtypeStruct((_SC_NW, 16), jnp.float32),
            jax.ShapeDtypeStruct((_SC_NW, 16), jnp.float32),
        ],
        scratch_types=[
            pltpu.VMEM((_SC_CH,), jnp.float32),
            pltpu.VMEM((_SC_CH,), jnp.float32),
            pltpu.VMEM((_SC_CH,), jnp.float32),
            pltpu.VMEM((_SC_CH,), jnp.float32),
            pltpu.VMEM((_SC_CH,), jnp.int32),
            pltpu.VMEM((_SC_CH,), jnp.int32),
            pltpu.VMEM((16,), jnp.float32),
            pltpu.VMEM((16,), jnp.float32),
        ],
        mesh=plsc.VectorSubcoreMesh(core_axis_name="c", subcore_axis_name="s"),
    )
    return f(x_flat, t_flat, r_flat)


@jax.jit
def kernel(input, target, ROI):
    num_sc, den_sc = _sc_partial(
        input.reshape(-1), target.reshape(-1), ROI.reshape(-1)
    )
    num_tc, den_tc = _tc_partial(input, target, ROI, _SC_NB)
    num = num_tc + jnp.sum(num_sc)
    den = den_tc + jnp.sum(den_sc)
    return num / den


# speedup vs baseline: 1.2186x; 1.2186x over previous
import jax
import jax.numpy as jnp
from jax import lax
from jax.experimental import pallas as pl
from jax.experimental.pallas import tpu as pltpu
from jax.experimental.pallas import tpu_sc as plsc

_NLO = 1.0000500033334732e-04
_NHI = 9.210340371976184
_LN2 = 0.6931471805599453

_LOGC = (
    -0.006151470821350813, 0.08406148105859756, -0.5094411969184875,
    1.8016612529754639, -4.118063449859619, 6.3783440589904785,
    -6.894534587860107, 5.635407447814941, -2.371283531188965,
)

_PIX = 512 * 512
_SC_NB = 1
_SC_NC, _SC_NS = 2, 16
_SC_NW = _SC_NC * _SC_NS
_SC_CH = 4096


def _tc_body(x_ref, t_ref, roi_ref, out_ref, acc_ref):
    i = pl.program_id(0)

    @pl.when(i == 0)
    def _init():
        acc_ref[0] = 0.0
        acc_ref[1] = 0.0

    x = x_ref[...]
    x0, x1, x2, x3 = x[:, 0], x[:, 1], x[:, 2], x[:, 3]
    s = (jnp.exp(x0) + jnp.exp(x1)) + (jnp.exp(x2) + jnp.exp(x3))

    t = t_ref[...]
    xt = jnp.where(t == 0, x0, jnp.where(t == 1, x1, jnp.where(t == 2, x2, x3)))
    nll = jnp.clip(jnp.log(s) - xt, _NLO, _NHI)

    live = roi_ref[...] != 0
    acc_ref[0] += jnp.sum(jnp.where(live, nll, 0.0))
    acc_ref[1] += jnp.sum(jnp.where(live, 1.0, 0.0))

    @pl.when(i == pl.num_programs(0) - 1)
    def _fin():
        out_ref[0, 0] = acc_ref[0]
        out_ref[0, 1] = acc_ref[1]


def _tc_partial(input, target, ROI, nb_skip):
    B, C, H, W = input.shape
    nrem = B - nb_skip
    NB = 2 if nrem % 2 == 0 else 1
    grid = (nrem // NB,)
    off = nb_skip // NB
    out = pl.pallas_call(
        _tc_body,
        grid=grid,
        in_specs=[
            pl.BlockSpec((NB, C, H, W), lambda i: (i + off, 0, 0, 0)),
            pl.BlockSpec((NB, H, W), lambda i: (i + off, 0, 0)),
            pl.BlockSpec((NB, H, W), lambda i: (i + off, 0, 0)),
        ],
        out_specs=pl.BlockSpec((1, 2), lambda i: (0, 0), memory_space=pltpu.SMEM),
        out_shape=jax.ShapeDtypeStruct((1, 2), jnp.float32),
        scratch_shapes=[pltpu.SMEM((2,), jnp.float32)],
    )(input, target, ROI)
    return out[0, 0], out[0, 1]


def _sc_body(x_hbm, t_hbm, r_hbm, num_out, den_out,
             x0b, x1b, x2b, x3b, tb, rb, nacc, dacc):
    wid = lax.axis_index("s") * _SC_NC + lax.axis_index("c")
    pix_per_w = _SC_NB * _PIX // _SC_NW
    n_chunks = pix_per_w // _SC_CH

    def step(i, acc):
        an, ad = acc
        sl = pl.ds(i * 16, 16)
        x0 = x0b[sl]
        x1 = x1b[sl]
        x2 = x2b[sl]
        x3 = x3b[sl]
        s = (jnp.exp(x0) + jnp.exp(x1)) + (jnp.exp(x2) + jnp.exp(x3))
        bits = lax.bitcast_convert_type(s, jnp.int32)
        e = jnp.right_shift(bits, 23) - 127
        mb = jnp.bitwise_or(jnp.bitwise_and(bits, 0x007FFFFF), 0x3F800000)
        m = lax.bitcast_convert_type(mb, jnp.float32)
        p = jnp.full((16,), _LOGC[0], jnp.float32)
        for c in _LOGC[1:]:
            p = p * m + c
        logs = e.astype(jnp.float32) * _LN2 + p
        t = tb[sl]
        xt = jnp.where(t == 0, x0, jnp.where(t == 1, x1, jnp.where(t == 2, x2, x3)))
        nll = jnp.clip(logs - xt, _NLO, _NHI)
        live = rb[sl] != 0
        an = an + jnp.where(live, nll, 0.0)
        ad = ad + jnp.where(live, 1.0, 0.0)
        return (an, ad)

    acc = (jnp.zeros((16,), jnp.float32), jnp.zeros((16,), jnp.float32))
    for j in range(n_chunks):
        gp = wid * pix_per_w + j * _SC_CH
        b = gp // _PIX
        poff = gp - b * _PIX
        xbase = b * (4 * _PIX) + poff
        pltpu.sync_copy(x_hbm.at[pl.ds(xbase, _SC_CH)], x0b)
        pltpu.sync_copy(x_hbm.at[pl.ds(xbase + _PIX, _SC_CH)], x1b)
        pltpu.sync_copy(x_hbm.at[pl.ds(xbase + 2 * _PIX, _SC_CH)], x2b)
        pltpu.sync_copy(x_hbm.at[pl.ds(xbase + 3 * _PIX, _SC_CH)], x3b)
        pltpu.sync_copy(t_hbm.at[pl.ds(gp, _SC_CH)], tb)
        pltpu.sync_copy(r_hbm.at[pl.ds(gp, _SC_CH)], rb)
        acc = lax.fori_loop(0, _SC_CH // 16, step, acc)

    nacc[...] = acc[0]
    dacc[...] = acc[1]
    pltpu.sync_copy(nacc, num_out.at[wid])
    pltpu.sync_copy(dacc, den_out.at[wid])


def _sc_partial(x_flat, t_flat, r_flat):
    f = pl.kernel(
        _sc_body,
        out_type=[
            jax.ShapeDtypeStruct((_SC_NW, 16), jnp.float32),
            jax.ShapeDtypeStruct((_SC_NW, 16), jnp.float32),
        ],
        scratch_types=[
            pltpu.VMEM((_SC_CH,), jnp.float32),
            pltpu.VMEM((_SC_CH,), jnp.float32),
            pltpu.VMEM((_SC_CH,), jnp.float32),
            pltpu.VMEM((_SC_CH,), jnp.float32),
            pltpu.VMEM((_SC_CH,), jnp.int32),
            pltpu.VMEM((_SC_CH,), jnp.int32),
            pltpu.VMEM((16,), jnp.float32),
            pltpu.VMEM((16,), jnp.float32),
        ],
        mesh=plsc.VectorSubcoreMesh(core_axis_name="c", subcore_axis_name="s"),
    )
    return f(x_flat, t_flat, r_flat)


@jax.jit
def kernel(input, target, ROI):
    num_sc, den_sc = _sc_partial(
        input.reshape(-1), target.reshape(-1), ROI.reshape(-1)
    )
    num_tc, den_tc = _tc_partial(input, target, ROI, _SC_NB)
    num = num_tc + jnp.sum(num_sc)
    den = den_tc + jnp.sum(den_sc)
    return num / den


# speedup vs baseline: 2.1486x; 1.7631x over previous
import jax
import jax.numpy as jnp
from jax import lax
from jax.experimental import pallas as pl
from jax.experimental.pallas import tpu as pltpu
from jax.experimental.pallas import tpu_sc as plsc

_NLO = 1.0000500033334732e-04
_NHI = 9.210340371976184
_LN2 = 0.6931471805599453

_LOGC = (
    -0.006151470821350813, 0.08406148105859756, -0.5094411969184875,
    1.8016612529754639, -4.118063449859619, 6.3783440589904785,
    -6.894534587860107, 5.635407447814941, -2.371283531188965,
)

_PIX = 512 * 512
_SC_NB = 2
_SC_NC, _SC_NS = 2, 16
_SC_NW = _SC_NC * _SC_NS
_SC_CH = 4096


def _tc_body(x_ref, t_ref, roi_ref, out_ref, acc_ref):
    i = pl.program_id(0)

    @pl.when(i == 0)
    def _init():
        acc_ref[0] = 0.0
        acc_ref[1] = 0.0

    x = x_ref[...]
    x0, x1, x2, x3 = x[:, 0], x[:, 1], x[:, 2], x[:, 3]
    s = (jnp.exp(x0) + jnp.exp(x1)) + (jnp.exp(x2) + jnp.exp(x3))

    t = t_ref[...]
    xt = jnp.where(t == 0, x0, jnp.where(t == 1, x1, jnp.where(t == 2, x2, x3)))
    nll = jnp.clip(jnp.log(s) - xt, _NLO, _NHI)

    live = roi_ref[...] != 0
    acc_ref[0] += jnp.sum(jnp.where(live, nll, 0.0))
    acc_ref[1] += jnp.sum(jnp.where(live, 1.0, 0.0))

    @pl.when(i == pl.num_programs(0) - 1)
    def _fin():
        out_ref[0, 0] = acc_ref[0]
        out_ref[0, 1] = acc_ref[1]


def _tc_partial(input, target, ROI, nb_skip):
    B, C, H, W = input.shape
    nrem = B - nb_skip
    NB = 2 if nrem % 2 == 0 else 1
    grid = (nrem // NB,)
    off = nb_skip // NB
    out = pl.pallas_call(
        _tc_body,
        grid=grid,
        in_specs=[
            pl.BlockSpec((NB, C, H, W), lambda i: (i + off, 0, 0, 0)),
            pl.BlockSpec((NB, H, W), lambda i: (i + off, 0, 0)),
            pl.BlockSpec((NB, H, W), lambda i: (i + off, 0, 0)),
        ],
        out_specs=pl.BlockSpec((1, 2), lambda i: (0, 0), memory_space=pltpu.SMEM),
        out_shape=jax.ShapeDtypeStruct((1, 2), jnp.float32),
        scratch_shapes=[pltpu.SMEM((2,), jnp.float32)],
    )(input, target, ROI)
    return out[0, 0], out[0, 1]


def _sc_body(x_hbm, t_hbm, r_hbm, num_out, den_out,
             x0b, x1b, x2b, x3b, tb, rb, nacc, dacc):
    wid = lax.axis_index("s") * _SC_NC + lax.axis_index("c")
    pix_per_w = _SC_NB * _PIX // _SC_NW
    n_chunks = pix_per_w // _SC_CH

    def step(i, acc):
        an, ad = acc
        sl = pl.ds(i * 16, 16)
        x0 = x0b[sl]
        x1 = x1b[sl]
        x2 = x2b[sl]
        x3 = x3b[sl]
        s = (jnp.exp(x0) + jnp.exp(x1)) + (jnp.exp(x2) + jnp.exp(x3))
        bits = lax.bitcast_convert_type(s, jnp.int32)
        e = jnp.right_shift(bits, 23) - 127
        mb = jnp.bitwise_or(jnp.bitwise_and(bits, 0x007FFFFF), 0x3F800000)
        m = lax.bitcast_convert_type(mb, jnp.float32)
        p = jnp.full((16,), _LOGC[0], jnp.float32)
        for c in _LOGC[1:]:
            p = p * m + c
        logs = e.astype(jnp.float32) * _LN2 + p
        t = tb[sl]
        xt = jnp.where(t == 0, x0, jnp.where(t == 1, x1, jnp.where(t == 2, x2, x3)))
        nll = jnp.clip(logs - xt, _NLO, _NHI)
        live = rb[sl] != 0
        an = an + jnp.where(live, nll, 0.0)
        ad = ad + jnp.where(live, 1.0, 0.0)
        return (an, ad)

    acc = (jnp.zeros((16,), jnp.float32), jnp.zeros((16,), jnp.float32))
    for j in range(n_chunks):
        gp = wid * pix_per_w + j * _SC_CH
        b = gp // _PIX
        poff = gp - b * _PIX
        xbase = b * (4 * _PIX) + poff
        pltpu.sync_copy(x_hbm.at[pl.ds(xbase, _SC_CH)], x0b)
        pltpu.sync_copy(x_hbm.at[pl.ds(xbase + _PIX, _SC_CH)], x1b)
        pltpu.sync_copy(x_hbm.at[pl.ds(xbase + 2 * _PIX, _SC_CH)], x2b)
        pltpu.sync_copy(x_hbm.at[pl.ds(xbase + 3 * _PIX, _SC_CH)], x3b)
        pltpu.sync_copy(t_hbm.at[pl.ds(gp, _SC_CH)], tb)
        pltpu.sync_copy(r_hbm.at[pl.ds(gp, _SC_CH)], rb)
        acc = lax.fori_loop(0, _SC_CH // 16, step, acc)

    nacc[...] = acc[0]
    dacc[...] = acc[1]
    pltpu.sync_copy(nacc, num_out.at[wid])
    pltpu.sync_copy(dacc, den_out.at[wid])


def _sc_partial(x_flat, t_flat, r_flat):
    f = pl.kernel(
        _sc_body,
        out_type=[
            jax.ShapeDtypeStruct((_SC_NW, 16), jnp.float32),
            jax.ShapeDtypeStruct((_SC_NW, 16), jnp.float32),
        ],
        scratch_types=[
            pltpu.VMEM((_SC_CH,), jnp.float32),
            pltpu.VMEM((_SC_CH,), jnp.float32),
            pltpu.VMEM((_SC_CH,), jnp.float32),
            pltpu.VMEM((_SC_CH,), jnp.float32),
            pltpu.VMEM((_SC_CH,), jnp.int32),
            pltpu.VMEM((_SC_CH,), jnp.int32),
            pltpu.VMEM((16,), jnp.float32),
            pltpu.VMEM((16,), jnp.float32),
        ],
        mesh=plsc.VectorSubcoreMesh(core_axis_name="c", subcore_axis_name="s"),
    )
    return f(x_flat, t_flat, r_flat)


@jax.jit
def kernel(input, target, ROI):
    num_sc, den_sc = _sc_partial(
        input[:_SC_NB].reshape(-1),
        target[:_SC_NB].reshape(-1),
        ROI[:_SC_NB].reshape(-1),
    )
    num_tc, den_tc = _tc_partial(input, target, ROI, _SC_NB)
    num = num_tc + jnp.sum(num_sc)
    den = den_tc + jnp.sum(den_sc)
    return num / den


# speedup vs baseline: 2.6991x; 1.2562x over previous
import jax
import jax.numpy as jnp
from jax import lax
from jax.experimental import pallas as pl
from jax.experimental.pallas import tpu as pltpu
from jax.experimental.pallas import tpu_sc as plsc

_NLO = 1.0000500033334732e-04
_NHI = 9.210340371976184
_LN2 = 0.6931471805599453

_LOGC = (
    -0.006151470821350813, 0.08406148105859756, -0.5094411969184875,
    1.8016612529754639, -4.118063449859619, 6.3783440589904785,
    -6.894534587860107, 5.635407447814941, -2.371283531188965,
)

_PIX = 512 * 512
_SC_NB = 2
_SC_NC, _SC_NS = 2, 16
_SC_NW = _SC_NC * _SC_NS
_SC_CH = 4096


def _tc_body(x_ref, t_ref, roi_ref, out_ref, acc_ref):
    i = pl.program_id(0)

    @pl.when(i == 0)
    def _init():
        acc_ref[0] = 0.0
        acc_ref[1] = 0.0

    x = x_ref[...]
    x0, x1, x2, x3 = x[:, 0], x[:, 1], x[:, 2], x[:, 3]
    s = (jnp.exp(x0) + jnp.exp(x1)) + (jnp.exp(x2) + jnp.exp(x3))

    t = t_ref[...]
    xt = jnp.where(t == 0, x0, jnp.where(t == 1, x1, jnp.where(t == 2, x2, x3)))
    nll = jnp.clip(jnp.log(s) - xt, _NLO, _NHI)

    live = roi_ref[...] != 0
    acc_ref[0] += jnp.sum(jnp.where(live, nll, 0.0))
    acc_ref[1] += jnp.sum(jnp.where(live, 1.0, 0.0))

    @pl.when(i == pl.num_programs(0) - 1)
    def _fin():
        out_ref[0, 0] = acc_ref[0]
        out_ref[0, 1] = acc_ref[1]


def _tc_partial(input, target, ROI, nb_skip):
    B, C, H, W = input.shape
    nrem = B - nb_skip
    NB = 2 if nrem % 2 == 0 else 1
    grid = (nrem // NB,)
    off = nb_skip // NB
    out = pl.pallas_call(
        _tc_body,
        grid=grid,
        in_specs=[
            pl.BlockSpec((NB, C, H, W), lambda i: (i + off, 0, 0, 0)),
            pl.BlockSpec((NB, H, W), lambda i: (i + off, 0, 0)),
            pl.BlockSpec((NB, H, W), lambda i: (i + off, 0, 0)),
        ],
        out_specs=pl.BlockSpec((1, 2), lambda i: (0, 0), memory_space=pltpu.SMEM),
        out_shape=jax.ShapeDtypeStruct((1, 2), jnp.float32),
        scratch_shapes=[pltpu.SMEM((2,), jnp.float32)],
    )(input, target, ROI)
    return out[0, 0], out[0, 1]


def _sc_body(x_hbm, t_hbm, r_hbm, num_out, den_out,
             x0b, x1b, x2b, x3b, tb, rb, nacc, dacc):
    wid = lax.axis_index("s") * _SC_NC + lax.axis_index("c")
    n_chunks = _SC_NB * 64 // _SC_NW

    def step(i, acc):
        an, ad = acc
        r = i // 32
        sl = pl.ds((i % 32) * 16, 16)
        x0 = x0b[r, sl]
        x1 = x1b[r, sl]
        x2 = x2b[r, sl]
        x3 = x3b[r, sl]
        s = (jnp.exp(x0) + jnp.exp(x1)) + (jnp.exp(x2) + jnp.exp(x3))
        bits = lax.bitcast_convert_type(s, jnp.int32)
        e = jnp.right_shift(bits, 23) - 127
        mb = jnp.bitwise_or(jnp.bitwise_and(bits, 0x007FFFFF), 0x3F800000)
        m = lax.bitcast_convert_type(mb, jnp.float32)
        p = jnp.full((16,), _LOGC[0], jnp.float32)
        for c in _LOGC[1:]:
            p = p * m + c
        logs = e.astype(jnp.float32) * _LN2 + p
        t = tb[r, sl]
        xt = jnp.where(t == 0, x0, jnp.where(t == 1, x1, jnp.where(t == 2, x2, x3)))
        nll = jnp.clip(logs - xt, _NLO, _NHI)
        live = rb[r, sl] != 0
        an = an + jnp.where(live, nll, 0.0)
        ad = ad + jnp.where(live, 1.0, 0.0)
        return (an, ad)

    acc = (jnp.zeros((16,), jnp.float32), jnp.zeros((16,), jnp.float32))
    for j in range(n_chunks):
        chunk = wid * n_chunks + j
        b = chunk // 64
        r0 = (chunk % 64) * 8
        rows = pl.ds(r0, 8)
        pltpu.sync_copy(x_hbm.at[b, 0, rows, :], x0b)
        pltpu.sync_copy(x_hbm.at[b, 1, rows, :], x1b)
        pltpu.sync_copy(x_hbm.at[b, 2, rows, :], x2b)
        pltpu.sync_copy(x_hbm.at[b, 3, rows, :], x3b)
        pltpu.sync_copy(t_hbm.at[b, rows, :], tb)
        pltpu.sync_copy(r_hbm.at[b, rows, :], rb)
        acc = lax.fori_loop(0, 256, step, acc)

    nacc[...] = acc[0]
    dacc[...] = acc[1]
    pltpu.sync_copy(nacc, num_out.at[wid])
    pltpu.sync_copy(dacc, den_out.at[wid])


def _sc_partial(x, t, r):
    f = pl.kernel(
        _sc_body,
        out_type=[
            jax.ShapeDtypeStruct((_SC_NW, 16), jnp.float32),
            jax.ShapeDtypeStruct((_SC_NW, 16), jnp.float32),
        ],
        scratch_types=[
            pltpu.VMEM((8, 512), jnp.float32),
            pltpu.VMEM((8, 512), jnp.float32),
            pltpu.VMEM((8, 512), jnp.float32),
            pltpu.VMEM((8, 512), jnp.float32),
            pltpu.VMEM((8, 512), jnp.int32),
            pltpu.VMEM((8, 512), jnp.int32),
            pltpu.VMEM((16,), jnp.float32),
            pltpu.VMEM((16,), jnp.float32),
        ],
        mesh=plsc.VectorSubcoreMesh(core_axis_name="c", subcore_axis_name="s"),
    )
    return f(x, t, r)


@jax.jit
def kernel(input, target, ROI):
    num_sc, den_sc = _sc_partial(input, target, ROI)
    num_tc, den_tc = _tc_partial(input, target, ROI, _SC_NB)
    num = num_tc + jnp.sum(num_sc)
    den = den_tc + jnp.sum(den_sc)
    return num / den
